# Initial kernel scaffold; baseline (speedup 1.0000x reference)
import jax
import jax.numpy as jnp
from jax.experimental import pallas as pl


def kernel(x_node, edge_index_0, edge_index_1, enc_W, enc_b, conv_W, conv_b, dec_W, dec_b, satt_W, satt_b, q_W, lin_W, lin_b):
    raise NotImplementedError("write your pallas kernel here")


# speedup vs baseline: 11.0968x; 11.0968x over previous
import functools

import jax
import jax.numpy as jnp
from jax import lax
from jax.experimental import pallas as pl
from jax.experimental.pallas import tpu as pltpu
from jax.experimental.pallas import tpu_sc as plsc

N = 10000
E = 320000
NTILES = 16
ROWS_PT = 632
NP = NTILES * ROWS_PT
EPT = E // NTILES
DK = 80
NCH = EPT // DK
BN = 1264
GRID = NP // BN

_mesh = functools.partial(
    plsc.VectorSubcoreMesh, core_axis_name="c", subcore_axis_name="s")


def _deg_body(d0, d1, ones_h, z_h, out, idx_v, ones_v, acc, sem):
    c = lax.axis_index("c")
    s = lax.axis_index("s")
    row0 = s * ROWS_PT
    pltpu.sync_copy(ones_h, ones_v)

    def run(dref, out_base):
        pltpu.sync_copy(z_h, acc.at[pl.ds(row0, ROWS_PT)])
        plsc.subcore_barrier()
        base = s * EPT

        def chunk(i, carry):
            off = base + i * DK
            pltpu.sync_copy(dref.at[pl.ds(off, DK)], idx_v)
            pltpu.sync_copy(ones_v, acc.at[idx_v], add=True)
            return carry

        lax.fori_loop(0, NCH, chunk, 0)
        plsc.subcore_barrier()
        pltpu.sync_copy(acc.at[pl.ds(row0, ROWS_PT)],
                        out.at[pl.ds(out_base + row0, ROWS_PT)])

    @pl.when(c == 0)
    def _():
        run(d0, 0)

    @pl.when(c == 1)
    def _():
        run(d1, NP)


_deg_call = pl.kernel(
    _deg_body,
    out_type=jax.ShapeDtypeStruct((2 * NP, 128), jnp.float32),
    mesh=_mesh(),
    scratch_types=[
        pltpu.VMEM((DK,), jnp.int32),
        pltpu.VMEM((DK, 128), jnp.float32),
        pltpu.VMEM_SHARED((NP, 128), jnp.float32),
        pltpu.SemaphoreType.DMA,
    ],
)


def _agg_body(u00, u01, u10, u11, s0, d0, s1, d1, z_h,
              o00, o01, o10, o11, idxs, idxd, rows, acc, sem):
    c = lax.axis_index("c")
    s = lax.axis_index("s")
    row0 = s * ROWS_PT

    def run(u, si, di, out):
        pltpu.sync_copy(z_h, acc.at[pl.ds(row0, ROWS_PT)])
        plsc.subcore_barrier()
        base = s * EPT

        def chunk(i, carry):
            off = base + i * DK
            pltpu.sync_copy(si.at[pl.ds(off, DK)], idxs)
            pltpu.sync_copy(di.at[pl.ds(off, DK)], idxd)
            pltpu.async_copy(u.at[idxs], rows, sem).wait()
            pltpu.sync_copy(rows, acc.at[idxd], add=True)
            return carry

        lax.fori_loop(0, NCH, chunk, 0)
        plsc.subcore_barrier()
        pltpu.sync_copy(acc.at[pl.ds(row0, ROWS_PT)],
                        out.at[pl.ds(row0, ROWS_PT)])
        plsc.subcore_barrier()

    @pl.when(c == 0)
    def _():
        run(u00, s0, d0, o00)
        run(u10, s1, d1, o10)

    @pl.when(c == 1)
    def _():
        run(u01, s0, d0, o01)
        run(u11, s1, d1, o11)


_agg_call = pl.kernel(
    _agg_body,
    out_type=[jax.ShapeDtypeStruct((NP, 128), jnp.float32)] * 4,
    mesh=_mesh(),
    scratch_types=[
        pltpu.VMEM((DK,), jnp.int32),
        pltpu.VMEM((DK,), jnp.int32),
        pltpu.VMEM((DK, 128), jnp.float32),
        pltpu.VMEM_SHARED((NP, 128), jnp.float32),
        pltpu.SemaphoreType.DMA,
    ],
)


def _enc_body(x_ref, w_ref, b_ref, deg_ref,
              u00, u01, u10, u11, di_ref):
    hh = jnp.dot(x_ref[...], w_ref[...], preferred_element_type=jnp.float32)
    hh = hh + b_ref[...]
    hh = jnp.where(hh > 0, hh, 0.1 * hh)
    dinv = lax.rsqrt(deg_ref[...] + 1.0)
    di_ref[...] = dinv
    u00[...] = hh[:, 0:128] * dinv[:, 0:1]
    u01[...] = hh[:, 128:256] * dinv[:, 0:1]
    u10[...] = hh[:, 256:384] * dinv[:, 1:2]
    u11[...] = hh[:, 384:512] * dinv[:, 1:2]


def _enc_call(xp, wcat, bcat, degcol):
    outs = [jax.ShapeDtypeStruct((NP, 128), jnp.float32)] * 4 + [
        jax.ShapeDtypeStruct((NP, 2), jnp.float32)]
    row = pl.BlockSpec((BN, 128), lambda i: (i, 0))
    return pl.pallas_call(
        _enc_body,
        grid=(GRID,),
        in_specs=[
            row,
            pl.BlockSpec((128, 512), lambda i: (0, 0)),
            pl.BlockSpec((1, 512), lambda i: (0, 0)),
            pl.BlockSpec((BN, 2), lambda i: (i, 0)),
        ],
        out_specs=[row, row, row, row,
                   pl.BlockSpec((BN, 2), lambda i: (i, 0))],
        out_shape=outs,
    )(xp, wcat, bcat, degcol)


def _mix_body(s00, s01, s10, s11, u00, u01, u10, u11, di_ref,
              cw_ref, cb_ref, dw_ref, db_ref, sw_ref, sb_ref, qt_ref, lw_ref,
              y0_ref, y1_ref, bs_ref):
    i = pl.program_id(0)
    S = (s00, s01, s10, s11)
    U = (u00, u01, u10, u11)
    di = di_ref[...]
    rowid = i * BN + lax.broadcasted_iota(jnp.int32, (BN, 1), 0)
    valid = rowid < N

    @pl.when(i == 0)
    def _():
        bs_ref[...] = jnp.zeros_like(bs_ref)

    bsl = []
    for m in range(2):
        dm = di[:, m:m + 1]
        heads = []
        for h in range(2):
            k = 2 * m + h
            cin = (S[k][...] + U[k][...]) * dm
            a = jnp.dot(cin, cw_ref[k], preferred_element_type=jnp.float32)
            a = a + cb_ref[k]
            a = jnp.where(a > 0, a, 0.1 * a)
            heads.append(
                jnp.dot(a, dw_ref[k], preferred_element_type=jnp.float32)
                + db_ref[k])
        z = jnp.concatenate(heads, axis=1)
        y = jnp.dot(z, lw_ref[...], preferred_element_type=jnp.float32)
        if m == 0:
            y0_ref[...] = y
        else:
            y1_ref[...] = y
        t = jnp.tanh(
            jnp.dot(z, sw_ref[...], preferred_element_type=jnp.float32)
            + sb_ref[...])
        bet = jnp.sum(t * qt_ref[...], axis=1, keepdims=True)
        bsl.append(jnp.sum(jnp.where(valid, bet, 0.0)))

    r8 = lax.broadcasted_iota(jnp.int32, (8, 128), 0)
    c128 = lax.broadcasted_iota(jnp.int32, (8, 128), 1)
    upd = jnp.where((r8 == 0) & (c128 == 0), bsl[0], 0.0) + \
        jnp.where((r8 == 1) & (c128 == 0), bsl[1], 0.0)
    bs_ref[...] = bs_ref[...] + upd


def _mix_call(Ss, Us, di2, cw, cb, dw, db, sw, sb, qt, lw):
    row = pl.BlockSpec((BN, 128), lambda i: (i, 0))
    full = lambda shp: pl.BlockSpec(shp, lambda i: tuple(0 for _ in shp))
    return pl.pallas_call(
        _mix_body,
        grid=(GRID,),
        in_specs=[row] * 8 + [
            pl.BlockSpec((BN, 2), lambda i: (i, 0)),
            full((4, 128, 128)), full((4, 1, 128)),
            full((4, 128, 64)), full((4, 1, 64)),
            full((128, 64)), full((1, 64)), full((1, 64)),
            full((128, 64)),
        ],
        out_specs=[
            pl.BlockSpec((BN, 64), lambda i: (i, 0)),
            pl.BlockSpec((BN, 64), lambda i: (i, 0)),
            pl.BlockSpec((8, 128), lambda i: (0, 0)),
        ],
        out_shape=[
            jax.ShapeDtypeStruct((NP, 64), jnp.float32),
            jax.ShapeDtypeStruct((NP, 64), jnp.float32),
            jax.ShapeDtypeStruct((8, 128), jnp.float32),
        ],
    )(*Ss, *Us, di2, cw, cb, dw, db, sw, sb, qt, lw)


def _fin_body(y0_ref, y1_ref, bs_ref, lb_ref, lp_ref, w_ref):
    b = bs_ref[:, 0:1] * (1.0 / N)
    r8 = lax.broadcasted_iota(jnp.int32, (8, 1), 0)
    vm = r8 < 2
    bmax = jnp.max(jnp.where(vm, b, -jnp.inf))
    e = jnp.where(vm, jnp.exp(b - bmax), 0.0)
    w = e / jnp.sum(e)
    w0 = w[0:1, 0:1]
    w1 = w[1:2, 0:1]
    t = w0 * y0_ref[...] + w1 * y1_ref[...] + lb_ref[...]
    tmax = jnp.max(t, axis=1, keepdims=True)
    lse = jnp.log(jnp.sum(jnp.exp(t - tmax), axis=1, keepdims=True))
    lp_ref[...] = t - tmax - lse
    c128 = lax.broadcasted_iota(jnp.int32, (8, 128), 1)
    w_ref[...] = jnp.where(c128 == 0, w, 0.0)


def _fin_call(y0, y1, bs, lb):
    return pl.pallas_call(
        _fin_body,
        grid=(GRID,),
        in_specs=[
            pl.BlockSpec((BN, 64), lambda i: (i, 0)),
            pl.BlockSpec((BN, 64), lambda i: (i, 0)),
            pl.BlockSpec((8, 128), lambda i: (0, 0)),
            pl.BlockSpec((1, 64), lambda i: (0, 0)),
        ],
        out_specs=[
            pl.BlockSpec((BN, 64), lambda i: (i, 0)),
            pl.BlockSpec((8, 128), lambda i: (0, 0)),
        ],
        out_shape=[
            jax.ShapeDtypeStruct((NP, 64), jnp.float32),
            jax.ShapeDtypeStruct((8, 128), jnp.float32),
        ],
    )(y0, y1, bs, lb)


def kernel(x_node, edge_index_0, edge_index_1, enc_W, enc_b, conv_W, conv_b,
           dec_W, dec_b, satt_W, satt_b, q_W, lin_W, lin_b):
    f32 = jnp.float32
    s0 = edge_index_0[0].astype(jnp.int32)
    d0 = edge_index_0[1].astype(jnp.int32)
    s1 = edge_index_1[0].astype(jnp.int32)
    d1 = edge_index_1[1].astype(jnp.int32)

    ones_h = jnp.ones((DK, 128), f32)
    z128 = jnp.zeros((ROWS_PT, 128), f32)

    deg = _deg_call(d0, d1, ones_h, z128)
    degcol = jnp.stack([deg[:NP, 0], deg[NP:, 0]], axis=1)

    xp = jnp.pad(x_node, ((0, NP - N), (0, 0)))
    wcat = jnp.concatenate(
        [enc_W[m, h] for m in range(2) for h in range(2)], axis=1)
    bcat = jnp.concatenate(
        [enc_b[m, h] for m in range(2) for h in range(2)])[None, :]

    u00, u01, u10, u11, di2 = _enc_call(xp, wcat, bcat, degcol)

    Ss = _agg_call(u00, u01, u10, u11, s0, d0, s1, d1, z128)

    cw = conv_W.reshape(4, 128, 128)
    cb = conv_b.reshape(4, 1, 128)
    dw = dec_W.reshape(4, 128, 64)
    db = dec_b.reshape(4, 1, 64)
    y0, y1, bs = _mix_call(Ss, (u00, u01, u10, u11), di2, cw, cb, dw, db,
                           satt_W, satt_b[None, :], q_W.T, lin_W)

    lp, wfull = _fin_call(y0, y1, bs, lin_b[None, :])
    return lp[:N], wfull[:2, :1]
